# baseline (speedup 1.0000x reference)
import functools

import jax
import jax.numpy as jnp
from jax import lax
from jax.experimental import pallas as pl
from jax.experimental.pallas import tpu as pltpu
from jax.experimental.pallas import tpu_sc as plsc

N = 10000
F = 128
NNZ = 160000
NSUB = 16
NCORE = 2
CH = 64
NCHUNK = 160
E_SUB = CH * NCHUNK
NNZ_PAD = E_SUB * NSUB
E_W = NNZ_PAD // (NSUB * NCORE)
HIST_PAD = 10240
HSLC = HIST_PAD // NSUB
NPASS = 3
PROWS = 3456
NPAD = NPASS * PROWS
ACC_ROWS = PROWS + 8
STRIPE = PROWS // NSUB
ZROWS = 72
E_HSUB = NNZ_PAD // NSUB

_prec = lax.Precision.HIGHEST
_mesh = plsc.VectorSubcoreMesh(core_axis_name="c", subcore_axis_name="s")

_sc_params = pltpu.CompilerParams()
if "needs_layout_passes" in pltpu.CompilerParams.__dataclass_fields__:
    import dataclasses as _dc
    _sc_params = _dc.replace(_sc_params, needs_layout_passes=False)


def _dot(a, b):
    return jnp.dot(a, b, preferred_element_type=jnp.float32, precision=_prec)


def _k1_body(x_ref, he_ref, wlin_ref, w0_ref, wab_ref,
             u_ref, v_ref, a_ref, b_ref):
    xl = _dot(x_ref[...], wlin_ref[...])
    el = _dot(he_ref[...], wlin_ref[...])
    xu, xv = xl[:, :F], xl[:, F:]
    eu, ev = el[:, :F], el[:, F:]
    w0 = w0_ref[...]
    u_ref[...] = _dot(xu, w0)
    v_ref[...] = _dot(xv, w0)
    wab = wab_ref[...]
    a_ref[...] = _dot(0.5 * (xu + xv), wab)
    b_ref[...] = _dot(0.5 * (eu + ev), wab)


def _k2_body(o0_ref, o1_ref, w1_ref, u_ref, v_ref):
    w1 = w1_ref[...]
    h0 = o0_ref[...]
    h1 = o1_ref[...]
    h0 = jnp.where(h0 > 0, h0, jnp.exp(jnp.minimum(h0, 0.0)) - 1.0)
    h1 = jnp.where(h1 > 0, h1, jnp.exp(jnp.minimum(h1, 0.0)) - 1.0)
    u_ref[...] = _dot(h0, w1)
    v_ref[...] = _dot(h1, w1)


def _k3_body(p0_ref, p1_ref, w2a_ref, w2b_ref, y_ref):
    y_ref[...] = _dot(p0_ref[...], w2a_ref[...]) + _dot(p1_ref[...], w2b_ref[...])


_RB = 1000
_RBP = 1296


def _row_spec(width):
    return pl.BlockSpec((_RB, width), lambda i: (i, 0))


def _rowp_spec(width):
    return pl.BlockSpec((_RBP, width), lambda i: (i, 0))


def _full_spec(r, c):
    return pl.BlockSpec((r, c), lambda i: (0, 0))


def _deg_body(idx_hbm, hist_hbm, idx_v, hist_v):
    cj = lax.axis_index("c")
    s = lax.axis_index("s")
    pltpu.sync_copy(idx_hbm.at[pl.ds(cj * NNZ_PAD + s * E_HSUB, E_HSUB)], idx_v)

    zero16 = jnp.zeros((16,), jnp.float32)

    @pl.loop(0, HIST_PAD, step=16)
    def _(i):
        hist_v[pl.ds(i, 16)] = zero16

    ones16 = jnp.ones((16,), jnp.float32)
    iota = lax.iota(jnp.int32, 16)
    base = s * E_HSUB

    @pl.loop(0, E_HSUB, step=16)
    def _(i):
        valid = (iota + (base + i)) < NNZ
        plsc.addupdate_scatter(hist_v, [idx_v[pl.ds(i, 16)]], ones16,
                               mask=valid)

    pltpu.sync_copy(hist_v,
                    hist_hbm.at[pl.ds((cj * NSUB + s) * HIST_PAD, HIST_PAD)])


def _kd_body(hp_ref, dn_ref, be_ref):
    h = hp_ref[...]
    dn = jnp.sum(h[:NSUB], axis=0)
    be = jnp.sum(h[NSUB:], axis=0)
    dninv = jnp.where(dn > 0, 0.5 / dn, 0.0)
    binv = jnp.where(be > 0, 0.5 / be, 0.0)
    dn_ref[...] = jnp.broadcast_to(dninv[None, :], (8, dninv.shape[0]))
    be_ref[...] = jnp.broadcast_to(binv[None, :], (8, binv.shape[0]))


def _coef_body(rc_hbm, a_hbm, b_hbm, dninv_hbm, binv_hbm,
               cua_hbm, cva_hbm, cub_hbm, cvb_hbm,
               row_v, col_v, a_v, b_v, dn_v, bi_v,
               ca_v, sa_v, msa_v, cb_v, sb_v, msb_v):
    cj = lax.axis_index("c")
    s = lax.axis_index("s")
    w = s * NCORE + cj
    pltpu.sync_copy(rc_hbm.at[pl.ds(w * E_W, E_W)], row_v)
    pltpu.sync_copy(rc_hbm.at[pl.ds(NNZ_PAD + w * E_W, E_W)], col_v)
    pltpu.sync_copy(a_hbm, a_v)
    pltpu.sync_copy(b_hbm, b_v)
    pltpu.sync_copy(dninv_hbm, dn_v)
    pltpu.sync_copy(binv_hbm, bi_v)

    base = w * E_W
    iota = lax.iota(jnp.int32, 16)

    @pl.loop(0, E_W, step=16)
    def _(i):
        rv = row_v[pl.ds(i, 16)]
        cv = col_v[pl.ds(i, 16)]
        t = plsc.load_gather(a_v, [rv]) + plsc.load_gather(b_v, [cv])
        e = jnp.exp(2.0 * t)
        p = 1.0 - 2.0 / (e + 1.0)
        den = 1.0 + p * p
        valid = (iota + (base + i)) < NNZ
        c = jnp.where(valid, (p * p - 1.0) / den, 0.0)
        sg = jnp.where(valid, 2.0 * p / den, 0.0)
        f = plsc.load_gather(dn_v, [rv]) * plsc.load_gather(bi_v, [cv])
        cb = c * f
        sb = sg * f
        ca_v[pl.ds(i, 16)] = c
        sa_v[pl.ds(i, 16)] = sg
        msa_v[pl.ds(i, 16)] = -sg
        cb_v[pl.ds(i, 16)] = cb
        sb_v[pl.ds(i, 16)] = sb
        msb_v[pl.ds(i, 16)] = -sb

    pltpu.sync_copy(ca_v, cua_hbm.at[pl.ds(base, E_W)])
    pltpu.sync_copy(sa_v, cua_hbm.at[pl.ds(NNZ_PAD + base, E_W)])
    pltpu.sync_copy(msa_v, cva_hbm.at[pl.ds(base, E_W)])
    pltpu.sync_copy(ca_v, cva_hbm.at[pl.ds(NNZ_PAD + base, E_W)])
    pltpu.sync_copy(cb_v, cub_hbm.at[pl.ds(base, E_W)])
    pltpu.sync_copy(msb_v, cub_hbm.at[pl.ds(NNZ_PAD + base, E_W)])
    pltpu.sync_copy(sb_v, cvb_hbm.at[pl.ds(base, E_W)])
    pltpu.sync_copy(cb_v, cvb_hbm.at[pl.ds(NNZ_PAD + base, E_W)])


def _prop_body(u_hbm, v_hbm, src_hbm, dst_hbm, cu_hbm, cv_hbm,
               o0_hbm, o1_hbm,
               src_v, dst_v, cu_v, cv_v, ubuf, vbuf, wbuf, dl_v, zbuf, accS,
               w_hbm):
    cj = lax.axis_index("c")
    s = lax.axis_index("s")
    pltpu.sync_copy(src_hbm.at[s], src_v)
    pltpu.sync_copy(dst_hbm.at[s], dst_v)
    coff = cj * NNZ_PAD + s * E_SUB
    pltpu.sync_copy(cu_hbm.at[pl.ds(coff, E_SUB)], cu_v)
    pltpu.sync_copy(cv_hbm.at[pl.ds(coff, E_SUB)], cv_v)

    zero16 = jnp.zeros((16,), jnp.float32)

    @pl.loop(0, ZROWS)
    def _(r):
        for f0 in range(F // 16):
            zbuf[r, pl.ds(f0 * 16, 16)] = zero16

    for p in range(NPASS):
        lo = p * PROWS
        for blk in range(STRIPE // ZROWS):
            pltpu.sync_copy(zbuf, accS.at[pl.ds(s * STRIPE + blk * ZROWS,
                                                ZROWS)])
        plsc.subcore_barrier()

        @pl.loop(0, NCHUNK)
        def _(ch):
            if p == 0:
                pltpu.sync_copy(u_hbm.at[src_v.at[ch]], ubuf)
                pltpu.sync_copy(v_hbm.at[src_v.at[ch]], vbuf)

                @pl.loop(0, CH)
                def _(e):
                    g = jnp.full((16,), ch * CH + e, dtype=jnp.int32)
                    cu_b = plsc.load_gather(cu_v, [g])
                    cv_b = plsc.load_gather(cv_v, [g])
                    for f0 in range(F // 16):
                        sl = pl.ds(f0 * 16, 16)
                        wbuf[e, sl] = cu_b * ubuf[e, sl] + cv_b * vbuf[e, sl]

                pltpu.sync_copy(wbuf, w_hbm.at[pl.ds(coff + ch * CH, CH)])
            else:
                pltpu.sync_copy(w_hbm.at[pl.ds(coff + ch * CH, CH)], wbuf)

            for q in range(CH // 16):
                dvec = dst_v[ch, pl.ds(q * 16, 16)]
                inb = (dvec >= lo) & (dvec < lo + PROWS)
                dl_v[pl.ds(q * 16, 16)] = jnp.where(inb, dvec - lo, PROWS)

            pltpu.sync_copy(wbuf, accS.at[dl_v], add=True)

        plsc.subcore_barrier()

        @pl.when(cj == 0)
        def _():
            pltpu.sync_copy(accS.at[pl.ds(s * STRIPE, STRIPE)],
                            o0_hbm.at[pl.ds(lo + s * STRIPE, STRIPE)])

        @pl.when(cj == 1)
        def _():
            pltpu.sync_copy(accS.at[pl.ds(s * STRIPE, STRIPE)],
                            o1_hbm.at[pl.ds(lo + s * STRIPE, STRIPE)])


def _propagate(u, v, src3, dst3, cu, cv):
    f32 = jnp.float32
    run = pl.kernel(
        _prop_body,
        mesh=_mesh,
        compiler_params=_sc_params,
        out_type=[jax.ShapeDtypeStruct((NPAD, F), f32),
                  jax.ShapeDtypeStruct((NPAD, F), f32)],
        scratch_types=[
            pltpu.VMEM((NCHUNK, CH), jnp.int32),
            pltpu.VMEM((NCHUNK, CH), jnp.int32),
            pltpu.VMEM((E_SUB,), f32),
            pltpu.VMEM((E_SUB,), f32),
            pltpu.VMEM((CH, F), f32),
            pltpu.VMEM((CH, F), f32),
            pltpu.VMEM((CH, F), f32),
            pltpu.VMEM((CH,), jnp.int32),
            pltpu.VMEM((ZROWS, F), f32),
            pltpu.VMEM_SHARED((ACC_ROWS, F), f32),
            pltpu.HBM((2 * NNZ_PAD, F), f32),
        ],
    )
    return run(u, v, src3, dst3, cu, cv)


def kernel(x, edge_index, hyperedge_attr, W_lin, W_sheaf, W_conv0, W_conv1,
           W_lin2):
    f32 = jnp.float32
    i32 = jnp.int32

    row = edge_index[0].astype(i32)
    col = edge_index[1].astype(i32)
    pad = NNZ_PAD - NNZ
    row_p = jnp.pad(row, (0, pad))
    col_p = jnp.pad(col, (0, pad))
    rc_flat = jnp.concatenate([row_p, col_p])

    w1 = W_sheaf[:F, :]
    w2 = W_sheaf[F:, :]
    wab = jnp.concatenate([w1, w2], axis=1)
    wab = jnp.pad(wab, ((0, 0), (0, F - 2)))
    w2a = W_lin2[:F, :]
    w2b = W_lin2[F:, :]

    k1 = pl.pallas_call(
        _k1_body,
        grid=(N // _RB,),
        in_specs=[_row_spec(F), _row_spec(F), _full_spec(F, 2 * F),
                  _full_spec(F, F), _full_spec(F, F)],
        out_specs=[_row_spec(F), _row_spec(F), _row_spec(F), _row_spec(F)],
        out_shape=[jax.ShapeDtypeStruct((N, F), f32)] * 4,
    )
    U0, V0, A_, B_ = k1(x, hyperedge_attr, W_lin, W_conv0, wab)
    a = A_[:, 0]
    b = B_[:, 1]

    deg = pl.kernel(
        _deg_body,
        mesh=_mesh,
        compiler_params=_sc_params,
        out_type=jax.ShapeDtypeStruct((2 * NSUB * HIST_PAD,), f32),
        scratch_types=[
            pltpu.VMEM((E_HSUB,), i32),
            pltpu.VMEM((HIST_PAD,), f32),
        ],
    )
    hist = deg(rc_flat)
    kd = pl.pallas_call(
        _kd_body,
        grid=(HIST_PAD // 2048,),
        in_specs=[pl.BlockSpec((2 * NSUB, 2048), lambda i: (0, i))],
        out_specs=[pl.BlockSpec((8, 2048), lambda i: (0, i))] * 2,
        out_shape=[jax.ShapeDtypeStruct((8, HIST_PAD), f32)] * 2,
    )
    dninv8, binv8 = kd(hist.reshape(2 * NSUB, HIST_PAD))
    dninv = dninv8[0]
    binv = binv8[0]

    coef = pl.kernel(
        _coef_body,
        mesh=_mesh,
        compiler_params=_sc_params,
        out_type=[jax.ShapeDtypeStruct((2 * NNZ_PAD,), f32)] * 4,
        scratch_types=[
            pltpu.VMEM((E_W,), i32),
            pltpu.VMEM((E_W,), i32),
            pltpu.VMEM((N,), f32),
            pltpu.VMEM((N,), f32),
            pltpu.VMEM((HIST_PAD,), f32),
            pltpu.VMEM((HIST_PAD,), f32),
            pltpu.VMEM((E_W,), f32),
            pltpu.VMEM((E_W,), f32),
            pltpu.VMEM((E_W,), f32),
            pltpu.VMEM((E_W,), f32),
            pltpu.VMEM((E_W,), f32),
            pltpu.VMEM((E_W,), f32),
        ],
    )
    cua, cva, cub, cvb = coef(rc_flat, a, b, dninv, binv)

    k2 = pl.pallas_call(
        _k2_body,
        grid=(NPAD // _RBP,),
        in_specs=[_rowp_spec(F), _rowp_spec(F), _full_spec(F, F)],
        out_specs=[_rowp_spec(F), _rowp_spec(F)],
        out_shape=[jax.ShapeDtypeStruct((NPAD, F), f32)] * 2,
    )

    U0p = jnp.pad(U0, ((0, NPAD - N), (0, 0)))
    V0p = jnp.pad(V0, ((0, NPAD - N), (0, 0)))
    src_row = row_p.reshape(NSUB, NCHUNK, CH)
    src_col = col_p.reshape(NSUB, NCHUNK, CH)
    src_stack = jnp.stack([src_row, src_col, src_row, src_col])
    dst_stack = jnp.stack([src_col, src_row, src_col, src_row])
    cu_stack = jnp.stack([cua, cub, cua, cub])
    cv_stack = jnp.stack([cva, cvb, cva, cvb])
    mid = jnp.array([False, True, False, False])

    def body(carry, xs):
        u, v = carry
        src_i, dst_i, cu_i, cv_i, mid_i = xs
        o0, o1 = _propagate(u, v, src_i, dst_i, cu_i, cv_i)
        t0, t1 = k2(o0, o1, W_conv1)
        u_n = jnp.where(mid_i, t0, o0)
        v_n = jnp.where(mid_i, t1, o1)
        return (u_n, v_n), 0
    (P0, P1), _ = lax.scan(
        body, (U0p, V0p), (src_stack, dst_stack, cu_stack, cv_stack, mid))

    nc = W_lin2.shape[1]
    k3 = pl.pallas_call(
        _k3_body,
        grid=(NPAD // _RBP,),
        in_specs=[_rowp_spec(F), _rowp_spec(F), _full_spec(F, nc),
                  _full_spec(F, nc)],
        out_specs=[_rowp_spec(nc)],
        out_shape=[jax.ShapeDtypeStruct((NPAD, nc), f32)],
    )
    (y,) = k3(P0, P1, w2a, w2b)
    return y[:N]


# speedup vs baseline: 1.2134x; 1.2134x over previous
import functools

import jax
import jax.numpy as jnp
from jax import lax
from jax.experimental import pallas as pl
from jax.experimental.pallas import tpu as pltpu
from jax.experimental.pallas import tpu_sc as plsc

N = 10000
F = 128
NNZ = 160000
NSUB = 16
NCORE = 2
CH = 128
NCHUNK = 80
E_SUB = CH * NCHUNK
NNZ_PAD = E_SUB * NSUB
E_W = NNZ_PAD // (NSUB * NCORE)
HIST_PAD = 10240
HSLC = HIST_PAD // NSUB
NPASS = 3
PROWS = 3456
NPAD = NPASS * PROWS
ACC_ROWS = PROWS + 8
STRIPE = PROWS // NSUB
ZROWS = 72
E_HSUB = NNZ_PAD // NSUB

_prec = lax.Precision.HIGHEST
_mesh = plsc.VectorSubcoreMesh(core_axis_name="c", subcore_axis_name="s")

_sc_params = pltpu.CompilerParams()
if "needs_layout_passes" in pltpu.CompilerParams.__dataclass_fields__:
    import dataclasses as _dc
    _sc_params = _dc.replace(_sc_params, needs_layout_passes=False)


def _dot(a, b):
    return jnp.dot(a, b, preferred_element_type=jnp.float32, precision=_prec)


def _k1_body(x_ref, he_ref, wlin_ref, w0_ref, wab_ref,
             u_ref, v_ref, a_ref, b_ref):
    xl = _dot(x_ref[...], wlin_ref[...])
    el = _dot(he_ref[...], wlin_ref[...])
    xu, xv = xl[:, :F], xl[:, F:]
    eu, ev = el[:, :F], el[:, F:]
    w0 = w0_ref[...]
    u_ref[...] = _dot(xu, w0)
    v_ref[...] = _dot(xv, w0)
    wab = wab_ref[...]
    a_ref[...] = _dot(0.5 * (xu + xv), wab)
    b_ref[...] = _dot(0.5 * (eu + ev), wab)


def _k2_body(o0_ref, o1_ref, w1_ref, u_ref, v_ref):
    w1 = w1_ref[...]
    h0 = o0_ref[...]
    h1 = o1_ref[...]
    h0 = jnp.where(h0 > 0, h0, jnp.exp(jnp.minimum(h0, 0.0)) - 1.0)
    h1 = jnp.where(h1 > 0, h1, jnp.exp(jnp.minimum(h1, 0.0)) - 1.0)
    u_ref[...] = _dot(h0, w1)
    v_ref[...] = _dot(h1, w1)


def _k3_body(p0_ref, p1_ref, w2a_ref, w2b_ref, y_ref):
    y_ref[...] = _dot(p0_ref[...], w2a_ref[...]) + _dot(p1_ref[...], w2b_ref[...])


_RB = 1000
_RBP = 1296


def _row_spec(width):
    return pl.BlockSpec((_RB, width), lambda i: (i, 0))


def _rowp_spec(width):
    return pl.BlockSpec((_RBP, width), lambda i: (i, 0))


def _full_spec(r, c):
    return pl.BlockSpec((r, c), lambda i: (0, 0))


def _deg_body(idx_hbm, hist_hbm, idx_v, hist_v):
    cj = lax.axis_index("c")
    s = lax.axis_index("s")
    pltpu.sync_copy(idx_hbm.at[pl.ds(cj * NNZ_PAD + s * E_HSUB, E_HSUB)], idx_v)

    zero16 = jnp.zeros((16,), jnp.float32)

    @pl.loop(0, HIST_PAD, step=16)
    def _(i):
        hist_v[pl.ds(i, 16)] = zero16

    ones16 = jnp.ones((16,), jnp.float32)
    iota = lax.iota(jnp.int32, 16)
    base = s * E_HSUB

    @pl.loop(0, E_HSUB, step=16)
    def _(i):
        valid = (iota + (base + i)) < NNZ
        plsc.addupdate_scatter(hist_v, [idx_v[pl.ds(i, 16)]], ones16,
                               mask=valid)

    pltpu.sync_copy(hist_v,
                    hist_hbm.at[pl.ds((cj * NSUB + s) * HIST_PAD, HIST_PAD)])


def _kd_body(hp_ref, dn_ref, be_ref):
    h = hp_ref[...]
    dn = jnp.sum(h[:NSUB], axis=0)
    be = jnp.sum(h[NSUB:], axis=0)
    dninv = jnp.where(dn > 0, 0.5 / dn, 0.0)
    binv = jnp.where(be > 0, 0.5 / be, 0.0)
    dn_ref[...] = jnp.broadcast_to(dninv[None, :], (8, dninv.shape[0]))
    be_ref[...] = jnp.broadcast_to(binv[None, :], (8, binv.shape[0]))


def _coef_body(rc_hbm, a_hbm, b_hbm, dninv_hbm, binv_hbm,
               cua_hbm, cva_hbm, cub_hbm, cvb_hbm,
               row_v, col_v, a_v, b_v, dn_v, bi_v,
               ca_v, sa_v, msa_v, cb_v, sb_v, msb_v):
    cj = lax.axis_index("c")
    s = lax.axis_index("s")
    w = s * NCORE + cj
    pltpu.sync_copy(rc_hbm.at[pl.ds(w * E_W, E_W)], row_v)
    pltpu.sync_copy(rc_hbm.at[pl.ds(NNZ_PAD + w * E_W, E_W)], col_v)
    pltpu.sync_copy(a_hbm, a_v)
    pltpu.sync_copy(b_hbm, b_v)
    pltpu.sync_copy(dninv_hbm, dn_v)
    pltpu.sync_copy(binv_hbm, bi_v)

    base = w * E_W
    iota = lax.iota(jnp.int32, 16)

    @pl.loop(0, E_W, step=16)
    def _(i):
        rv = row_v[pl.ds(i, 16)]
        cv = col_v[pl.ds(i, 16)]
        t = plsc.load_gather(a_v, [rv]) + plsc.load_gather(b_v, [cv])
        e = jnp.exp(2.0 * t)
        p = 1.0 - 2.0 / (e + 1.0)
        den = 1.0 + p * p
        valid = (iota + (base + i)) < NNZ
        c = jnp.where(valid, (p * p - 1.0) / den, 0.0)
        sg = jnp.where(valid, 2.0 * p / den, 0.0)
        f = plsc.load_gather(dn_v, [rv]) * plsc.load_gather(bi_v, [cv])
        cb = c * f
        sb = sg * f
        ca_v[pl.ds(i, 16)] = c
        sa_v[pl.ds(i, 16)] = sg
        msa_v[pl.ds(i, 16)] = -sg
        cb_v[pl.ds(i, 16)] = cb
        sb_v[pl.ds(i, 16)] = sb
        msb_v[pl.ds(i, 16)] = -sb

    pltpu.sync_copy(ca_v, cua_hbm.at[pl.ds(base, E_W)])
    pltpu.sync_copy(sa_v, cua_hbm.at[pl.ds(NNZ_PAD + base, E_W)])
    pltpu.sync_copy(msa_v, cva_hbm.at[pl.ds(base, E_W)])
    pltpu.sync_copy(ca_v, cva_hbm.at[pl.ds(NNZ_PAD + base, E_W)])
    pltpu.sync_copy(cb_v, cub_hbm.at[pl.ds(base, E_W)])
    pltpu.sync_copy(msb_v, cub_hbm.at[pl.ds(NNZ_PAD + base, E_W)])
    pltpu.sync_copy(sb_v, cvb_hbm.at[pl.ds(base, E_W)])
    pltpu.sync_copy(cb_v, cvb_hbm.at[pl.ds(NNZ_PAD + base, E_W)])


def _prop_body(u_hbm, v_hbm, src_hbm, dst_hbm, cu_hbm, cv_hbm,
               o0_hbm, o1_hbm,
               src_v, dst_v, cu_v, cv_v, ubuf, vbuf, wbuf, dl_v, zbuf, accS,
               w_hbm):
    cj = lax.axis_index("c")
    s = lax.axis_index("s")
    pltpu.sync_copy(src_hbm.at[s], src_v)
    pltpu.sync_copy(dst_hbm.at[s], dst_v)
    coff = cj * NNZ_PAD + s * E_SUB
    pltpu.sync_copy(cu_hbm.at[pl.ds(coff, E_SUB)], cu_v)
    pltpu.sync_copy(cv_hbm.at[pl.ds(coff, E_SUB)], cv_v)

    zero16 = jnp.zeros((16,), jnp.float32)

    @pl.loop(0, ZROWS)
    def _(r):
        for f0 in range(F // 16):
            zbuf[r, pl.ds(f0 * 16, 16)] = zero16

    for p in range(NPASS):
        lo = p * PROWS
        for blk in range(STRIPE // ZROWS):
            pltpu.sync_copy(zbuf, accS.at[pl.ds(s * STRIPE + blk * ZROWS,
                                                ZROWS)])
        plsc.subcore_barrier()

        @pl.loop(0, NCHUNK)
        def _(ch):
            if p == 0:
                pltpu.sync_copy(u_hbm.at[src_v.at[ch]], ubuf)
                pltpu.sync_copy(v_hbm.at[src_v.at[ch]], vbuf)

                @pl.loop(0, CH)
                def _(e):
                    g = jnp.full((16,), ch * CH + e, dtype=jnp.int32)
                    cu_b = plsc.load_gather(cu_v, [g])
                    cv_b = plsc.load_gather(cv_v, [g])
                    for f0 in range(F // 16):
                        sl = pl.ds(f0 * 16, 16)
                        wbuf[e, sl] = cu_b * ubuf[e, sl] + cv_b * vbuf[e, sl]

                pltpu.sync_copy(wbuf, w_hbm.at[pl.ds(coff + ch * CH, CH)])
            else:
                pltpu.sync_copy(w_hbm.at[pl.ds(coff + ch * CH, CH)], wbuf)

            for q in range(CH // 16):
                dvec = dst_v[ch, pl.ds(q * 16, 16)]
                inb = (dvec >= lo) & (dvec < lo + PROWS)
                dl_v[pl.ds(q * 16, 16)] = jnp.where(inb, dvec - lo, PROWS)

            pltpu.sync_copy(wbuf, accS.at[dl_v], add=True)

        plsc.subcore_barrier()

        @pl.when(cj == 0)
        def _():
            pltpu.sync_copy(accS.at[pl.ds(s * STRIPE, STRIPE)],
                            o0_hbm.at[pl.ds(lo + s * STRIPE, STRIPE)])

        @pl.when(cj == 1)
        def _():
            pltpu.sync_copy(accS.at[pl.ds(s * STRIPE, STRIPE)],
                            o1_hbm.at[pl.ds(lo + s * STRIPE, STRIPE)])


def _propagate(u, v, src3, dst3, cu, cv):
    f32 = jnp.float32
    run = pl.kernel(
        _prop_body,
        mesh=_mesh,
        compiler_params=_sc_params,
        out_type=[jax.ShapeDtypeStruct((NPAD, F), f32),
                  jax.ShapeDtypeStruct((NPAD, F), f32)],
        scratch_types=[
            pltpu.VMEM((NCHUNK, CH), jnp.int32),
            pltpu.VMEM((NCHUNK, CH), jnp.int32),
            pltpu.VMEM((E_SUB,), f32),
            pltpu.VMEM((E_SUB,), f32),
            pltpu.VMEM((CH, F), f32),
            pltpu.VMEM((CH, F), f32),
            pltpu.VMEM((CH, F), f32),
            pltpu.VMEM((CH,), jnp.int32),
            pltpu.VMEM((ZROWS, F), f32),
            pltpu.VMEM_SHARED((ACC_ROWS, F), f32),
            pltpu.HBM((2 * NNZ_PAD, F), f32),
        ],
    )
    return run(u, v, src3, dst3, cu, cv)


def kernel(x, edge_index, hyperedge_attr, W_lin, W_sheaf, W_conv0, W_conv1,
           W_lin2):
    f32 = jnp.float32
    i32 = jnp.int32

    row = edge_index[0].astype(i32)
    col = edge_index[1].astype(i32)
    pad = NNZ_PAD - NNZ
    row_p = jnp.pad(row, (0, pad))
    col_p = jnp.pad(col, (0, pad))
    rc_flat = jnp.concatenate([row_p, col_p])

    w1 = W_sheaf[:F, :]
    w2 = W_sheaf[F:, :]
    wab = jnp.concatenate([w1, w2], axis=1)
    wab = jnp.pad(wab, ((0, 0), (0, F - 2)))
    w2a = W_lin2[:F, :]
    w2b = W_lin2[F:, :]

    k1 = pl.pallas_call(
        _k1_body,
        grid=(N // _RB,),
        in_specs=[_row_spec(F), _row_spec(F), _full_spec(F, 2 * F),
                  _full_spec(F, F), _full_spec(F, F)],
        out_specs=[_row_spec(F), _row_spec(F), _row_spec(F), _row_spec(F)],
        out_shape=[jax.ShapeDtypeStruct((N, F), f32)] * 4,
    )
    U0, V0, A_, B_ = k1(x, hyperedge_attr, W_lin, W_conv0, wab)
    a = A_[:, 0]
    b = B_[:, 1]

    deg = pl.kernel(
        _deg_body,
        mesh=_mesh,
        compiler_params=_sc_params,
        out_type=jax.ShapeDtypeStruct((2 * NSUB * HIST_PAD,), f32),
        scratch_types=[
            pltpu.VMEM((E_HSUB,), i32),
            pltpu.VMEM((HIST_PAD,), f32),
        ],
    )
    hist = deg(rc_flat)
    kd = pl.pallas_call(
        _kd_body,
        grid=(HIST_PAD // 2048,),
        in_specs=[pl.BlockSpec((2 * NSUB, 2048), lambda i: (0, i))],
        out_specs=[pl.BlockSpec((8, 2048), lambda i: (0, i))] * 2,
        out_shape=[jax.ShapeDtypeStruct((8, HIST_PAD), f32)] * 2,
    )
    dninv8, binv8 = kd(hist.reshape(2 * NSUB, HIST_PAD))
    dninv = dninv8[0]
    binv = binv8[0]

    coef = pl.kernel(
        _coef_body,
        mesh=_mesh,
        compiler_params=_sc_params,
        out_type=[jax.ShapeDtypeStruct((2 * NNZ_PAD,), f32)] * 4,
        scratch_types=[
            pltpu.VMEM((E_W,), i32),
            pltpu.VMEM((E_W,), i32),
            pltpu.VMEM((N,), f32),
            pltpu.VMEM((N,), f32),
            pltpu.VMEM((HIST_PAD,), f32),
            pltpu.VMEM((HIST_PAD,), f32),
            pltpu.VMEM((E_W,), f32),
            pltpu.VMEM((E_W,), f32),
            pltpu.VMEM((E_W,), f32),
            pltpu.VMEM((E_W,), f32),
            pltpu.VMEM((E_W,), f32),
            pltpu.VMEM((E_W,), f32),
        ],
    )
    cua, cva, cub, cvb = coef(rc_flat, a, b, dninv, binv)

    k2 = pl.pallas_call(
        _k2_body,
        grid=(NPAD // _RBP,),
        in_specs=[_rowp_spec(F), _rowp_spec(F), _full_spec(F, F)],
        out_specs=[_rowp_spec(F), _rowp_spec(F)],
        out_shape=[jax.ShapeDtypeStruct((NPAD, F), f32)] * 2,
    )

    U0p = jnp.pad(U0, ((0, NPAD - N), (0, 0)))
    V0p = jnp.pad(V0, ((0, NPAD - N), (0, 0)))
    src_row = row_p.reshape(NSUB, NCHUNK, CH)
    src_col = col_p.reshape(NSUB, NCHUNK, CH)
    src_stack = jnp.stack([src_row, src_col, src_row, src_col])
    dst_stack = jnp.stack([src_col, src_row, src_col, src_row])
    cu_stack = jnp.stack([cua, cub, cua, cub])
    cv_stack = jnp.stack([cva, cvb, cva, cvb])
    mid = jnp.array([False, True, False, False])

    def body(carry, xs):
        u, v = carry
        src_i, dst_i, cu_i, cv_i, mid_i = xs
        o0, o1 = _propagate(u, v, src_i, dst_i, cu_i, cv_i)
        t0, t1 = k2(o0, o1, W_conv1)
        u_n = jnp.where(mid_i, t0, o0)
        v_n = jnp.where(mid_i, t1, o1)
        return (u_n, v_n), 0
    (P0, P1), _ = lax.scan(
        body, (U0p, V0p), (src_stack, dst_stack, cu_stack, cv_stack, mid))

    nc = W_lin2.shape[1]
    k3 = pl.pallas_call(
        _k3_body,
        grid=(NPAD // _RBP,),
        in_specs=[_rowp_spec(F), _rowp_spec(F), _full_spec(F, nc),
                  _full_spec(F, nc)],
        out_specs=[_rowp_spec(nc)],
        out_shape=[jax.ShapeDtypeStruct((NPAD, nc), f32)],
    )
    (y,) = k3(P0, P1, w2a, w2b)
    return y[:N]
